# Initial kernel scaffold; baseline (speedup 1.0000x reference)
import jax
import jax.numpy as jnp
from jax.experimental import pallas as pl


def kernel(ligand_x, protein_x, params, ligand_edge_index, protein_edge_index, ligand_batch, protein_batch):
    raise NotImplementedError("write your pallas kernel here")


# speedup vs baseline: 2.8813x; 2.8813x over previous
import functools

import jax
import jax.numpy as jnp
from jax import lax
from jax.experimental import pallas as pl
from jax.experimental.pallas import tpu as pltpu
from jax.experimental.pallas import tpu_sc as plsc

HID = 256
NHEAD = 8
NLAYERS = 5
NSEG = 128
HALF = HID // 2

NC, NS = 2, 16
CH = 128
ACC_ROWS = 10240

NEG_INF = float("-inf")


@functools.lru_cache(maxsize=None)
def _make_edge_agg(n_nodes: int, e_pad: int):
    n_chunks = e_pad // (NS * CH)
    ept = n_chunks * CH
    zrows = ACC_ROWS // NS

    mesh = plsc.VectorSubcoreMesh(
        core_axis_name="c", subcore_axis_name="s", num_cores=NC,
        num_subcores=NS)

    @functools.partial(
        pl.kernel,
        out_type=jax.ShapeDtypeStruct((2 * ACC_ROWS, HALF), jnp.float32),
        mesh=mesh,
        scratch_types=[
            pltpu.VMEM((CH,), jnp.int32),
            pltpu.VMEM((CH,), jnp.int32),
            pltpu.VMEM((CH,), jnp.int32),
            pltpu.VMEM((CH, HALF), jnp.float32),
            pltpu.VMEM_SHARED((ACC_ROWS, HALF), jnp.float32),
            pltpu.SemaphoreType.DMA,
        ],
    )
    def edge_agg(h_hbm, src_hbm, dst_hbm, zeros_hbm, out_hbm,
                 srcv, srcv2, dstv, rowsv, acc_sh, sem):
        cid = lax.axis_index("c")
        sid = lax.axis_index("s")
        coff = cid * n_nodes

        pltpu.sync_copy(zeros_hbm.at[pl.ds(sid * zrows, zrows)],
                        acc_sh.at[pl.ds(sid * zrows, zrows)])
        plsc.subcore_barrier()

        @pl.loop(0, n_chunks)
        def _chunk(ci):
            off = sid * ept + ci * CH
            pltpu.sync_copy(src_hbm.at[pl.ds(off, CH)], srcv)
            pltpu.sync_copy(dst_hbm.at[pl.ds(off, CH)], dstv)
            for j in range(CH // 16):
                srcv2[pl.ds(j * 16, 16)] = srcv[pl.ds(j * 16, 16)] + coff
            pltpu.async_copy(h_hbm.at[srcv2], rowsv, sem).wait()
            pltpu.sync_copy(rowsv, acc_sh.at[dstv], add=True)

        plsc.subcore_barrier()
        pltpu.sync_copy(
            acc_sh.at[pl.ds(sid * zrows, zrows)],
            out_hbm.at[pl.ds(cid * ACC_ROWS + sid * zrows, zrows)])

    return edge_agg


def _edge_agg(h_split, src, dst, zeros, n_nodes):
    e = src.shape[0]
    e_pad = ((e + NS * CH - 1) // (NS * CH)) * (NS * CH)
    if e_pad != e:
        pad = e_pad - e
        src = jnp.concatenate([src, jnp.zeros((pad,), jnp.int32)])
        dst = jnp.concatenate([dst, jnp.full((pad,), n_nodes, jnp.int32)])
    return _make_edge_agg(n_nodes, e_pad)(h_split, src, dst, zeros)


def _mm(x, w):
    return lax.dot_general(x, w, (((1,), (1,)), ((), ())),
                           preferred_element_type=jnp.float32)


def _mmT(a, b):
    return lax.dot_general(a, b, (((1,), (0,)), ((), ())),
                           preferred_element_type=jnp.float32)


def _bn(x, g, b):
    m = jnp.mean(x, 0)
    v = jnp.var(x, 0)
    return g * (x - m) / jnp.sqrt(v + 1e-5) + b


def _ln(x, g, b):
    m = jnp.mean(x, -1, keepdims=True)
    v = jnp.var(x, -1, keepdims=True)
    return g * (x - m) / jnp.sqrt(v + 1e-5) + b


def _onehot_n(batch_col, n):
    i = lax.broadcasted_iota(jnp.int32, (n, NSEG), 1)
    return jnp.where(batch_col == i, 1.0, 0.0).astype(jnp.float32)


def _onehot_t(batch_row, n):
    i = lax.broadcasted_iota(jnp.int32, (NSEG, n), 0)
    return jnp.where(batch_row == i, 1.0, 0.0).astype(jnp.float32)


def _call(body, out_shapes, *args):
    return pl.pallas_call(body, out_shape=out_shapes)(*args)


def _k_embed(x_ref, w1, b1, g1, be1, w2, b2, out):
    h = _mm(x_ref[...], w1[...]) + b1[...]
    h = jax.nn.relu(_bn(h, g1[...], be1[...]))
    out[...] = _mm(h, w2[...]) + b2[...]


def _embed(x, p):
    n = x.shape[0]
    return _call(_k_embed, jax.ShapeDtypeStruct((n, HID), jnp.float32),
                 x, p['W1'], p['b1'], p['g1'], p['be1'], p['W2'], p['b2'])


def _k_virtual(h_ref, v_ref, bn_ref, bt_ref, w, b, g, be,
               hmid, hsplit, vnew):
    n = h_ref.shape[0]
    h = h_ref[...]
    ot = _onehot_t(bt_ref[...], n)
    sums = _mmT(ot, h)
    z = v_ref[...] + sums
    t = _mm(z, w[...]) + b[...]
    vn = jax.nn.relu(_bn(t, g[...], be[...]))
    vnew[...] = vn
    o = _onehot_n(bn_ref[...], n)
    hm = h + _mmT(o, vn)
    hmid[...] = hm
    hsplit[...] = jnp.concatenate([hm[:, :HALF], hm[:, HALF:]], axis=0)


def _virtual(h, v, batch_col, batch_row, p):
    n = h.shape[0]
    return _call(
        _k_virtual,
        [jax.ShapeDtypeStruct((n, HID), jnp.float32),
         jax.ShapeDtypeStruct((2 * n, HALF), jnp.float32),
         jax.ShapeDtypeStruct((NSEG, HID), jnp.float32)],
        h, v, batch_col, batch_row, p['W'], p['b'], p['g'], p['be'])


def _k_gin(h_ref, agg_ref, eps_ref, wm1, bm1, gm, bem, wm2, bm2, g, be, out):
    n = h_ref.shape[0]
    h = h_ref[...]
    agg = jnp.concatenate([agg_ref[:n, :], agg_ref[ACC_ROWS:ACC_ROWS + n, :]],
                          axis=1)
    z = (1.0 + eps_ref[0, 0]) * h + agg
    t = _mm(z, wm1[...]) + bm1[...]
    t = jax.nn.relu(_bn(t, gm[...], bem[...]))
    u = _mm(t, wm2[...]) + bm2[...]
    out[...] = h + jax.nn.relu(_bn(u, g[...], be[...]))


def _gin(h, agg, p):
    n = h.shape[0]
    eps = p['eps'].reshape(1, 1)
    return _call(_k_gin, jax.ShapeDtypeStruct((n, HID), jnp.float32),
                 h, agg, eps, p['Wm1'], p['bm1'], p['gm'], p['bem'],
                 p['Wm2'], p['bm2'], p['g'], p['be'])


def _k_mean(h_ref, bt_ref, mean_out):
    n = h_ref.shape[0]
    ot = _onehot_t(bt_ref[...], n)
    cnt = jnp.sum(ot, axis=1, keepdims=True)
    mean_out[...] = _mmT(ot, h_ref[...]) / jnp.maximum(cnt, 1.0)


def _seg_mean_k(h, batch_row):
    n = h.shape[0]
    return _call(_k_mean, jax.ShapeDtypeStruct((NSEG, HID), jnp.float32),
                 h, batch_row)


def _k_attn(lm_ref, pm_ref, wv_l, bv_l, wo_l, bo_l, wv_p, bv_p, wo_p, bo_p,
            gl, bl, gp, bp, lp_out, pp_out):
    lm, pm = lm_ref[...], pm_ref[...]
    la = _mm(_mm(pm, wv_l[...]) + bv_l[...], wo_l[...]) + bo_l[...]
    pa = _mm(_mm(lm, wv_p[...]) + bv_p[...], wo_p[...]) + bo_p[...]
    lp_out[...] = _ln(lm + la, gl[...], bl[...])
    pp_out[...] = _ln(pm + pa, gp[...], bp[...])


def _k_pool(h_ref, p_ref, bn_ref, bt_ref, w1, b1, w2, b2,
            mean_out, max_out, att_out):
    n = h_ref.shape[0]
    o = _onehot_n(bn_ref[...], n)
    h2 = h_ref[...] + _mmT(o, p_ref[...])
    ot = _onehot_t(bt_ref[...], n)
    cnt = jnp.sum(ot, axis=1, keepdims=True)
    mean_out[...] = _mmT(ot, h2) / jnp.maximum(cnt, 1.0)
    t = jnp.tanh(_mm(h2, w1[...]) + b1[...])
    s = jnp.sum(t * w2[...], axis=1, keepdims=True) + b2[0, 0]
    smax = jnp.max(jnp.where(o > 0.0, s, NEG_INF), axis=0)
    e = jnp.exp(s[:, 0] - _mmT(o, smax.reshape(NSEG, 1))[:, 0])
    d = _mmT(ot, e.reshape(n, 1))
    w_row = e / _mmT(o, d)[:, 0]
    att_out[...] = _mmT(ot, h2 * w_row[:, None])
    b_col = bn_ref[...]

    def seg_body(si, _):
        m = jnp.where(b_col == si, h2, NEG_INF)
        max_out[pl.ds(si, 1), :] = jnp.max(m, axis=0, keepdims=True)
        return 0

    lax.fori_loop(0, NSEG, seg_body, 0)


def _pool(h, p, batch_col, batch_row, pp):
    n = h.shape[0]
    return _call(
        _k_pool,
        [jax.ShapeDtypeStruct((NSEG, HID), jnp.float32)] * 3,
        h, p, batch_col, batch_row, pp['W1'], pp['b1'], pp['W2'],
        pp['b2'].reshape(1, 1))


def _k_head(g_ref, w1, b1, g1, be1, w2, b2, g2, be2, w3, b3, g3, be3,
            w4, b4, w5, b5, out):
    h = jax.nn.relu(_ln(_mm(g_ref[...], w1[...]) + b1[...], g1[...], be1[...]))
    h = jax.nn.relu(_ln(_mm(h, w2[...]) + b2[...], g2[...], be2[...]))
    h = jax.nn.relu(_ln(_mm(h, w3[...]) + b3[...], g3[...], be3[...]))
    h = jax.nn.relu(_mm(h, w4[...]) + b4[...])
    out[...] = jnp.sum(h * w5[...], axis=1, keepdims=True) + b5[0, 0]


def kernel(ligand_x, protein_x, params, ligand_edge_index, protein_edge_index,
           ligand_batch, protein_batch):
    n_lig = ligand_x.shape[0]
    n_prot = protein_x.shape[0]
    zeros = jnp.zeros((ACC_ROWS, HALF), jnp.float32)

    lb_col = ligand_batch.reshape(n_lig, 1)
    lb_row = ligand_batch.reshape(1, n_lig)
    pb_col = protein_batch.reshape(n_prot, 1)
    pb_row = protein_batch.reshape(1, n_prot)

    lig_h = _embed(ligand_x, params['lig_embed'])
    prot_h = _embed(protein_x, params['prot_embed'])
    lig_v = jnp.broadcast_to(params['lig_virtual'], (NSEG, HID))
    prot_v = jnp.broadcast_to(params['prot_virtual'], (NSEG, HID))

    l_src, l_dst = ligand_edge_index[0], ligand_edge_index[1]
    p_src, p_dst = protein_edge_index[0], protein_edge_index[1]

    for i in range(NLAYERS):
        lp = params['lig_layers'][i]
        lig_h, lig_split, lig_v = _virtual(lig_h, lig_v, lb_col, lb_row,
                                           lp['vmlp'])
        lig_agg = _edge_agg(lig_split, l_src, l_dst, zeros, n_lig)
        lig_h = _gin(lig_h, lig_agg, lp['gin'])

        pp = params['prot_layers'][i]
        prot_h, prot_split, prot_v = _virtual(prot_h, prot_v, pb_col, pb_row,
                                              pp['vmlp'])
        prot_agg = _edge_agg(prot_split, p_src, p_dst, zeros, n_prot)
        prot_h = _gin(prot_h, prot_agg, pp['gin'])

    lig_mean = _seg_mean_k(lig_h, lb_row)
    prot_mean = _seg_mean_k(prot_h, pb_row)

    al, ap = params['attn_lig'], params['attn_prot']
    wv_l, bv_l = al['Win'][2 * HID:], al['bin'][2 * HID:]
    wv_p, bv_p = ap['Win'][2 * HID:], ap['bin'][2 * HID:]
    lig_p, prot_p = _call(
        _k_attn,
        [jax.ShapeDtypeStruct((NSEG, HID), jnp.float32)] * 2,
        lig_mean, prot_mean,
        wv_l, bv_l, al['Wout'], al['bout'],
        wv_p, bv_p, ap['Wout'], ap['bout'],
        params['norm_lig']['g'], params['norm_lig']['b'],
        params['norm_prot']['g'], params['norm_prot']['b'])

    lm, lx, latt = _pool(lig_h, lig_p, lb_col, lb_row, params['pool_lig'])
    pm, px, patt = _pool(prot_h, prot_p, pb_col, pb_row, params['pool_prot'])

    g = jnp.concatenate([lm, lx, latt, pm, px, patt], axis=-1)
    p = params['pred']
    return _call(_k_head, jax.ShapeDtypeStruct((NSEG, 1), jnp.float32),
                 g, p['W1'], p['b1'], p['g1'], p['be1'],
                 p['W2'], p['b2'], p['g2'], p['be2'],
                 p['W3'], p['b3'], p['g3'], p['be3'],
                 p['W4'], p['b4'], p['W5'], p['b5'].reshape(1, 1))
